# baseline (speedup 1.0000x reference)
import jax
import jax.numpy as jnp
from jax.experimental import pallas as pl

_N = 5000
_C = 81


def _body(x_ref, lab_ref, w_ref, pcp_ref, imgw_ref, iml_ref, out_ref):
    x = x_ref[...]
    rows = jax.lax.broadcasted_iota(jnp.int32, (648, 8), 0)
    cols = jax.lax.broadcasted_iota(jnp.int32, (648, 8), 1)
    sel = (rows == cols * 81).astype(jnp.float32)
    p8 = jax.lax.dot_general(
        x, sel, (((1,), (0,)), ((), ())), preferred_element_type=jnp.float32
    )
    mask0 = lab_ref[...] == 0
    t0 = jnp.sum(jnp.where(mask0, w_ref[...] * jnp.log(p8), 0.0))
    term0 = jnp.where(iml_ref[0, 0] != 0, -t0, 0.0)

    q = pcp_ref[0, 0]
    c_idx = jax.lax.broadcasted_iota(jnp.int32, (1, _C), 1)
    mask1 = (c_idx >= 1) & (iml_ref[...] != 0) & (q == c_idx.astype(jnp.float32))
    term1 = -jnp.sum(jnp.where(mask1, imgw_ref[0, 0] * jnp.log(q), 0.0))

    out_ref[...] = jnp.reshape((term0 + term1) / jnp.float32(_N), (1, 1))


def kernel(pcl_prob, labels, cls_loss_weights, gt_assignment, pc_labels,
           pc_probs, pc_count, img_cls_loss_weights, im_labels_real):
    x = pcl_prob.reshape(625, 648)
    lab = labels.reshape(625, 8)
    w = cls_loss_weights.reshape(625, 8)
    out = pl.pallas_call(
        _body,
        out_shape=jax.ShapeDtypeStruct((1, 1), jnp.float32),
    )(x, lab, w, pc_probs, img_cls_loss_weights, im_labels_real)
    return out[0, 0]
